# baseline (speedup 1.0000x reference)
import functools
import jax
import jax.numpy as jnp
from jax import lax
from jax.experimental import pallas as pl
from jax.experimental.pallas import tpu as pltpu
from jax.experimental.pallas import tpu_sc as plsc

N = 10000
D = 128
HID = 256
NG = 16
E = 320000
ETOT = E + N
L = 16
NC, NS = 2, 16
NW = NC * NS
NPT = 320
NPTR_W = 336
NPTR_PAD = 31 * NPT + NPTR_W
SSW = 64
SS_PAD = ETOT + 2 * SSW + 16
NEG = -3.0e38

ROW_BLK = 1000


def _proj_body(x_ref, w_ref, a_ref, h_ref, sa_ref):
    h = jnp.dot(x_ref[...], w_ref[...], preferred_element_type=jnp.float32)
    h_ref[...] = h
    sa_ref[...] = jnp.dot(h, a_ref[...], preferred_element_type=jnp.float32)


def _gat_project(x, W, a_s, a_d, H):
    K = x.shape[1]
    HC = H * HID
    heads = jnp.arange(HC, dtype=jnp.int32) // HID
    chans = jnp.arange(HC, dtype=jnp.int32) % HID
    cols = jnp.arange(2 * H, dtype=jnp.int32)
    vals = jnp.concatenate([a_s, a_d], axis=0)
    A = jnp.where(heads[:, None] == cols[None, :] % H, vals.T[chans, :], 0.0)
    h, sa = pl.pallas_call(
        _proj_body,
        grid=(N // ROW_BLK,),
        in_specs=[
            pl.BlockSpec((ROW_BLK, K), lambda i: (i, 0)),
            pl.BlockSpec((K, HC), lambda i: (0, 0)),
            pl.BlockSpec((HC, 2 * H), lambda i: (0, 0)),
        ],
        out_specs=[
            pl.BlockSpec((ROW_BLK, HC), lambda i: (i, 0)),
            pl.BlockSpec((ROW_BLK, 2 * H), lambda i: (i, 0)),
        ],
        out_shape=[
            jax.ShapeDtypeStruct((N, HC), jnp.float32),
            jax.ShapeDtypeStruct((N, 2 * H), jnp.float32),
        ],
    )(x, W, A)
    return h, sa


def _sc_edge_body(h_hbm, sa_hbm, ss_hbm, nptr_hbm, b_hbm, out_hbm,
                  ssbuf, sabuf, rows0, rows1, accbuf, biasbuf, nptr_s,
                  sem_a, sem_b, *, H):
    HC = H * HID
    wid = lax.axis_index("s") * NC + lax.axis_index("c")
    pltpu.sync_copy(sa_hbm, sabuf)
    pltpu.sync_copy(b_hbm, biasbuf)
    pltpu.sync_copy(nptr_hbm.at[pl.ds(wid * NPT, NPTR_W)], nptr_s)
    n0 = wid * NPT
    nhi = jnp.minimum(n0 + NPT, N)
    lidx = lax.iota(jnp.int32, L)

    def node_body(d, _):
        i = d - n0
        pv = nptr_s[pl.ds(i, L)]
        p0 = pv[0]
        p1 = pv[1]
        ws0 = (p0 // 8) * 8
        nwin = (p1 - ws0 + SSW - 1) // SSW
        adb = [plsc.load_gather(
            sabuf, [jnp.broadcast_to(d * (2 * H) + H + hd, (L,))])
            for hd in range(H)]

        for k in range(HC // L):
            accbuf[pl.ds(k * L, L)] = jnp.zeros((L,), jnp.float32)

        def load_window(ws):
            pltpu.sync_copy(ss_hbm.at[pl.ds(ws, SSW)], ssbuf)

        def logits(gi0, ws):
            pos = gi0 + lidx
            valid = (pos >= p0) & (pos < p1)
            li = pos - ws
            srcv = plsc.load_gather(ssbuf, [li])
            es = []
            for hd in range(H):
                asv = plsc.load_gather(sabuf, [srcv * (2 * H) + hd])
                e = asv + adb[hd]
                e = jnp.where(e >= 0.0, e, 0.2 * e)
                es.append(e)
            return srcv, valid, es

        def win_ab(w, carry):
            ws = ws0 + w * SSW
            load_window(ws)
            ng = jnp.minimum((p1 - ws + L - 1) // L, SSW // L)

            def grp_ab(g, c):
                ms, ss_ = c
                gi0 = ws + g * L
                _, valid, es = logits(gi0, ws)
                ms2, ss2 = [], []
                for hd in range(H):
                    e = jnp.where(valid, es[hd], NEG)
                    m_new = jnp.maximum(ms[hd], e)
                    s_new = (ss_[hd] * jnp.exp(ms[hd] - m_new)
                             + jnp.where(valid, jnp.exp(e - m_new), 0.0))
                    ms2.append(m_new)
                    ss2.append(s_new)
                return (tuple(ms2), tuple(ss2))

            return lax.fori_loop(0, ng, grp_ab, carry)

        zero = jnp.zeros((L,), jnp.float32)
        init = (tuple(jnp.full((L,), NEG, jnp.float32) for _ in range(H)),
                tuple(zero for _ in range(H)))
        ms, ss_ = lax.fori_loop(0, nwin, win_ab, init)
        mb, db = [], []
        for hd in range(H):
            m = jnp.max(ms[hd])
            s = jnp.sum(ss_[hd] * jnp.exp(ms[hd] - jnp.broadcast_to(m, (L,))))
            mb.append(jnp.broadcast_to(m, (L,)))
            db.append(jnp.broadcast_to(s + 1e-16, (L,)))

        def rloop(rbuf, al):
            def rbody(q, _r):
                for u in range(4):
                    r = q * 4 + u
                    rv = jnp.broadcast_to(r, (L,))
                    abs_ = [al[hd].at[rv].get(mode="promise_in_bounds")
                            for hd in range(H)]
                    for hd in range(H):
                        for j in range(HID // L):
                            sl = pl.ds(hd * HID + j * L, L)
                            plsc.addupdate(accbuf.at[sl], rbuf[r, sl] * abs_[hd])
                return 0
            lax.fori_loop(0, L // 4, rbody, 0)

        def win_c(w, _c):
            ws = ws0 + w * SSW

            @pl.when(nwin > 1)
            def _():
                load_window(ws)

            ng = jnp.minimum((p1 - ws + L - 1) // L, SSW // L)
            npair = (ng + 1) // 2

            def prep(g):
                gi0 = ws + g * L
                srcv, valid, es = logits(gi0, ws)
                al = [jnp.where(valid, jnp.exp(es[hd] - mb[hd]) / db[hd], 0.0)
                      for hd in range(H)]
                return srcv, al

            def pair(k, _2):
                g1 = 2 * k + 1
                s0, a0 = prep(2 * k)
                s1, a1 = prep(g1)
                have1 = g1 < ng
                cp0 = pltpu.async_copy(h_hbm.at[s0], rows0, sem_a)

                @pl.when(have1)
                def _():
                    pltpu.async_copy(h_hbm.at[s1], rows1, sem_b)

                cp0.wait()
                rloop(rows0, a0)

                @pl.when(have1)
                def _():
                    pltpu.make_async_copy(h_hbm.at[s1], rows1, sem_b).wait()
                    rloop(rows1, a1)

                return 0

            lax.fori_loop(0, npair, pair, 0)
            return 0

        lax.fori_loop(0, nwin, win_c, 0)

        for k in range(HC // L):
            sl = pl.ds(k * L, L)
            accbuf[sl] = jnp.maximum(accbuf[sl] + biasbuf[sl], 0.0)
        pltpu.sync_copy(accbuf, out_hbm.at[d])
        return 0

    lax.fori_loop(n0, nhi, node_body, 0)


def _gat_edge_sc(h, sa, ss_pad, nptr_pad, b, H):
    HC = H * HID
    mesh = plsc.VectorSubcoreMesh(core_axis_name="c", subcore_axis_name="s")
    kfn = pl.kernel(
        functools.partial(_sc_edge_body, H=H),
        out_type=jax.ShapeDtypeStruct((N, HC), jnp.float32),
        mesh=mesh,
        compiler_params=pltpu.CompilerParams(needs_layout_passes=False),
        scratch_types=[
            pltpu.VMEM((SSW,), jnp.int32),
            pltpu.VMEM((N * 2 * H,), jnp.float32),
            pltpu.VMEM((L, HC), jnp.float32),
            pltpu.VMEM((L, HC), jnp.float32),
            pltpu.VMEM((HC,), jnp.float32),
            pltpu.VMEM((HC,), jnp.float32),
            pltpu.VMEM((NPTR_W,), jnp.int32),
            pltpu.SemaphoreType.DMA,
            pltpu.SemaphoreType.DMA,
        ],
    )
    return kfn(h, sa.reshape(-1), ss_pad, nptr_pad, b)


def _pool_body(batch_ref, h_ref, pooled_ref):
    b = jnp.broadcast_to(batch_ref[0:1, :], (NG, N))
    g = lax.broadcasted_iota(jnp.int32, (NG, N), 0)
    P = (b == g).astype(jnp.float32)
    cnts = jnp.sum(P, axis=1)
    pooled_ref[...] = (jnp.dot(P, h_ref[...], preferred_element_type=jnp.float32)
                       / jnp.maximum(cnts, 1.0)[:, None])


def _pool(batch, h):
    return pl.pallas_call(
        _pool_body,
        out_shape=jax.ShapeDtypeStruct((NG, HID), jnp.float32),
    )(jnp.broadcast_to(batch[None, :], (8, N)), h)


def _heads_body(p_ref, wa1_ref, ba1_ref, wa2_ref, ba2_ref,
                wc1_ref, bc1_ref, wc2_ref, bc2_ref,
                act_ref, val_ref):
    p = p_ref[...]
    za = jax.nn.relu(jnp.dot(p, wa1_ref[...], preferred_element_type=jnp.float32)
                     + ba1_ref[0, :])
    act_ref[...] = jnp.tanh(
        jnp.dot(za, wa2_ref[...], preferred_element_type=jnp.float32) + ba2_ref[0, :])

    @pl.when(pl.program_id(0) == 0)
    def _():
        zc = jax.nn.relu(jnp.dot(p, wc1_ref[...], preferred_element_type=jnp.float32)
                         + bc1_ref[0, :])
        val_ref[...] = (jnp.dot(zc, wc2_ref[...], preferred_element_type=jnp.float32)
                        + bc2_ref[0, :])


def _heads(pooled, Wa1, ba1, Wa2, ba2, Wc1, bc1, Wc2, bc2):
    NE = Wa2.shape[1]
    CBLK = 12800
    action, value = pl.pallas_call(
        _heads_body,
        grid=(NE // CBLK,),
        in_specs=[
            pl.BlockSpec((NG, HID), lambda j: (0, 0)),
            pl.BlockSpec((HID, HID), lambda j: (0, 0)),
            pl.BlockSpec((1, HID), lambda j: (0, 0)),
            pl.BlockSpec((HID, CBLK), lambda j: (0, j)),
            pl.BlockSpec((1, CBLK), lambda j: (0, j)),
            pl.BlockSpec((HID, HID), lambda j: (0, 0)),
            pl.BlockSpec((1, HID), lambda j: (0, 0)),
            pl.BlockSpec((HID, 8), lambda j: (0, 0)),
            pl.BlockSpec((1, 8), lambda j: (0, 0)),
        ],
        out_specs=[
            pl.BlockSpec((NG, CBLK), lambda j: (0, j)),
            pl.BlockSpec((NG, 8), lambda j: (0, 0)),
        ],
        out_shape=[
            jax.ShapeDtypeStruct((NG, NE), jnp.float32),
            jax.ShapeDtypeStruct((NG, 8), jnp.float32),
        ],
    )(pooled, Wa1, ba1[None, :], Wa2, ba2[None, :], Wc1, bc1[None, :],
      jnp.pad(Wc2, ((0, 0), (0, 7))), jnp.pad(bc2, (0, 7))[None, :])
    return action, value[:, :1]


def kernel(x, edge_index, batch, W1, a1s, a1d, b1, W2, a2s, a2d, b2,
           W3, a3s, a3d, b3, Wa1, ba1, Wa2, ba2, Wc1, bc1, Wc2, bc2):
    loop = jnp.arange(N, dtype=edge_index.dtype)
    src = jnp.concatenate([edge_index[0], loop])
    dst = jnp.concatenate([edge_index[1], loop])
    perm = jnp.argsort(dst)
    ss = src[perm].astype(jnp.int32)
    ds = dst[perm]
    nptr = jnp.searchsorted(ds, jnp.arange(N + 1, dtype=jnp.int32)).astype(jnp.int32)
    nptr_pad = jnp.concatenate(
        [nptr, jnp.full((NPTR_PAD - (N + 1),), ETOT, jnp.int32)])
    ss_pad = jnp.concatenate([ss, jnp.zeros((SS_PAD - ETOT,), jnp.int32)])

    h, sa = _gat_project(x, W1, a1s, a1d, 4)
    h = _gat_edge_sc(h, sa, ss_pad, nptr_pad, b1, 4)
    h, sa = _gat_project(h, W2, a2s, a2d, 4)
    h = _gat_edge_sc(h, sa, ss_pad, nptr_pad, b2, 4)
    h, sa = _gat_project(h, W3, a3s, a3d, 1)
    h = _gat_edge_sc(h, sa, ss_pad, nptr_pad, b3, 1)

    pooled = _pool(batch, h)
    return _heads(pooled, Wa1, ba1, Wa2, ba2, Wc1, bc1, Wc2, bc2)


# speedup vs baseline: 1.0286x; 1.0286x over previous
import functools
import jax
import jax.numpy as jnp
from jax import lax
from jax.experimental import pallas as pl
from jax.experimental.pallas import tpu as pltpu
from jax.experimental.pallas import tpu_sc as plsc

N = 10000
D = 128
HID = 256
NG = 16
E = 320000
ETOT = E + N
L = 16
NC, NS = 2, 16
NW = NC * NS
NPT = 320
NPTR_W = 336
NPTR_PAD = 31 * NPT + NPTR_W
SSW = 64
SS_PAD = ETOT + 2 * SSW + 16
NEG = -3.0e38

ROW_BLK = 1000


def _proj_body(x_ref, w_ref, a_ref, h_ref, sa_ref):
    h = jnp.dot(x_ref[...], w_ref[...], preferred_element_type=jnp.float32)
    h_ref[...] = h
    sa_ref[...] = jnp.dot(h, a_ref[...], preferred_element_type=jnp.float32)


def _gat_project(x, W, a_s, a_d, H):
    K = x.shape[1]
    HC = H * HID
    heads = jnp.arange(HC, dtype=jnp.int32) // HID
    chans = jnp.arange(HC, dtype=jnp.int32) % HID
    cols = jnp.arange(2 * H, dtype=jnp.int32)
    vals = jnp.concatenate([a_s, a_d], axis=0)
    A = jnp.where(heads[:, None] == cols[None, :] % H, vals.T[chans, :], 0.0)
    h, sa = pl.pallas_call(
        _proj_body,
        grid=(N // ROW_BLK,),
        in_specs=[
            pl.BlockSpec((ROW_BLK, K), lambda i: (i, 0)),
            pl.BlockSpec((K, HC), lambda i: (0, 0)),
            pl.BlockSpec((HC, 2 * H), lambda i: (0, 0)),
        ],
        out_specs=[
            pl.BlockSpec((ROW_BLK, HC), lambda i: (i, 0)),
            pl.BlockSpec((ROW_BLK, 2 * H), lambda i: (i, 0)),
        ],
        out_shape=[
            jax.ShapeDtypeStruct((N, HC), jnp.float32),
            jax.ShapeDtypeStruct((N, 2 * H), jnp.float32),
        ],
    )(x, W, A)
    return h, sa


def _sc_edge_body(h_hbm, sa_hbm, ss_hbm, nptr_hbm, b_hbm, out_hbm,
                  ssbuf, sabuf, rows0, rows1, accbuf, biasbuf, nptr_s,
                  sem_a, sem_b, *, H):
    HC = H * HID
    wid = lax.axis_index("s") * NC + lax.axis_index("c")
    pltpu.sync_copy(sa_hbm, sabuf)
    pltpu.sync_copy(b_hbm, biasbuf)
    pltpu.sync_copy(nptr_hbm.at[pl.ds(wid * NPT, NPTR_W)], nptr_s)
    n0 = wid * NPT
    nhi = jnp.minimum(n0 + NPT, N)
    lidx = lax.iota(jnp.int32, L)

    def node_body(d, _):
        i = d - n0
        pv = nptr_s[pl.ds(i, L)]
        p0 = pv[0]
        p1 = pv[1]
        ws0 = (p0 // 8) * 8
        nwin = (p1 - ws0 + SSW - 1) // SSW
        adb = [plsc.load_gather(
            sabuf, [jnp.broadcast_to(d * (2 * H) + H + hd, (L,))])
            for hd in range(H)]

        for k in range(HC // L):
            accbuf[pl.ds(k * L, L)] = jnp.zeros((L,), jnp.float32)

        def load_window(ws):
            pltpu.sync_copy(ss_hbm.at[pl.ds(ws, SSW)], ssbuf)

        def logits(gi0, ws):
            pos = gi0 + lidx
            valid = (pos >= p0) & (pos < p1)
            li = pos - ws
            srcv = plsc.load_gather(ssbuf, [li])
            es = []
            for hd in range(H):
                asv = plsc.load_gather(sabuf, [srcv * (2 * H) + hd])
                e = asv + adb[hd]
                e = jnp.where(e >= 0.0, e, 0.2 * e)
                es.append(e)
            return srcv, valid, es

        def win_ab(w, carry):
            ws = ws0 + w * SSW
            load_window(ws)
            ng = jnp.minimum((p1 - ws + L - 1) // L, SSW // L)

            def grp_ab(g, c):
                ms, ss_ = c
                gi0 = ws + g * L
                _, valid, es = logits(gi0, ws)
                ms2, ss2 = [], []
                for hd in range(H):
                    e = jnp.where(valid, es[hd], NEG)
                    m_new = jnp.maximum(ms[hd], e)
                    s_new = (ss_[hd] * jnp.exp(ms[hd] - m_new)
                             + jnp.where(valid, jnp.exp(e - m_new), 0.0))
                    ms2.append(m_new)
                    ss2.append(s_new)
                return (tuple(ms2), tuple(ss2))

            return lax.fori_loop(0, ng, grp_ab, carry)

        zero = jnp.zeros((L,), jnp.float32)
        init = (tuple(jnp.full((L,), NEG, jnp.float32) for _ in range(H)),
                tuple(zero for _ in range(H)))
        ms, ss_ = lax.fori_loop(0, nwin, win_ab, init)
        mb, db = [], []
        for hd in range(H):
            m = jnp.max(ms[hd])
            s = jnp.sum(ss_[hd] * jnp.exp(ms[hd] - jnp.broadcast_to(m, (L,))))
            mb.append(jnp.broadcast_to(m, (L,)))
            db.append(jnp.broadcast_to(s + 1e-16, (L,)))

        def rloop(rbuf, al):
            def rbody(r, _r):
                rv = jnp.broadcast_to(r, (L,))
                for hd in range(H):
                    ab = al[hd].at[rv].get(mode="promise_in_bounds")
                    for j in range(HID // L):
                        sl = pl.ds(hd * HID + j * L, L)
                        plsc.addupdate(accbuf.at[sl], rbuf[r, sl] * ab)
                return 0
            lax.fori_loop(0, L, rbody, 0)

        def win_c(w, _c):
            ws = ws0 + w * SSW

            @pl.when(nwin > 1)
            def _():
                load_window(ws)

            ng = jnp.minimum((p1 - ws + L - 1) // L, SSW // L)
            npair = (ng + 1) // 2

            def prep(g):
                gi0 = ws + g * L
                srcv, valid, es = logits(gi0, ws)
                al = [jnp.where(valid, jnp.exp(es[hd] - mb[hd]) / db[hd], 0.0)
                      for hd in range(H)]
                return srcv, al

            def pair(k, _2):
                g1 = 2 * k + 1
                s0, a0 = prep(2 * k)
                s1, a1 = prep(g1)
                have1 = g1 < ng
                cp0 = pltpu.async_copy(h_hbm.at[s0], rows0, sem_a)

                @pl.when(have1)
                def _():
                    pltpu.async_copy(h_hbm.at[s1], rows1, sem_b)

                cp0.wait()
                rloop(rows0, a0)

                @pl.when(have1)
                def _():
                    pltpu.make_async_copy(h_hbm.at[s1], rows1, sem_b).wait()
                    rloop(rows1, a1)

                return 0

            lax.fori_loop(0, npair, pair, 0)
            return 0

        lax.fori_loop(0, nwin, win_c, 0)

        for k in range(HC // L):
            sl = pl.ds(k * L, L)
            accbuf[sl] = jnp.maximum(accbuf[sl] + biasbuf[sl], 0.0)
        pltpu.sync_copy(accbuf, out_hbm.at[d])
        return 0

    lax.fori_loop(n0, nhi, node_body, 0)


def _gat_edge_sc(h, sa, ss_pad, nptr_pad, b, H):
    HC = H * HID
    mesh = plsc.VectorSubcoreMesh(core_axis_name="c", subcore_axis_name="s")
    kfn = pl.kernel(
        functools.partial(_sc_edge_body, H=H),
        out_type=jax.ShapeDtypeStruct((N, HC), jnp.float32),
        mesh=mesh,
        compiler_params=pltpu.CompilerParams(needs_layout_passes=False),
        scratch_types=[
            pltpu.VMEM((SSW,), jnp.int32),
            pltpu.VMEM((N * 2 * H,), jnp.float32),
            pltpu.VMEM((L, HC), jnp.float32),
            pltpu.VMEM((L, HC), jnp.float32),
            pltpu.VMEM((HC,), jnp.float32),
            pltpu.VMEM((HC,), jnp.float32),
            pltpu.VMEM((NPTR_W,), jnp.int32),
            pltpu.SemaphoreType.DMA,
            pltpu.SemaphoreType.DMA,
        ],
    )
    return kfn(h, sa.reshape(-1), ss_pad, nptr_pad, b)


def _pool_body(batch_ref, h_ref, pooled_ref):
    b = jnp.broadcast_to(batch_ref[0:1, :], (NG, N))
    g = lax.broadcasted_iota(jnp.int32, (NG, N), 0)
    P = (b == g).astype(jnp.float32)
    cnts = jnp.sum(P, axis=1)
    pooled_ref[...] = (jnp.dot(P, h_ref[...], preferred_element_type=jnp.float32)
                       / jnp.maximum(cnts, 1.0)[:, None])


def _pool(batch, h):
    return pl.pallas_call(
        _pool_body,
        out_shape=jax.ShapeDtypeStruct((NG, HID), jnp.float32),
    )(jnp.broadcast_to(batch[None, :], (8, N)), h)


def _heads_body(p_ref, wa1_ref, ba1_ref, wa2_ref, ba2_ref,
                wc1_ref, bc1_ref, wc2_ref, bc2_ref,
                act_ref, val_ref):
    p = p_ref[...]
    za = jax.nn.relu(jnp.dot(p, wa1_ref[...], preferred_element_type=jnp.float32)
                     + ba1_ref[0, :])
    act_ref[...] = jnp.tanh(
        jnp.dot(za, wa2_ref[...], preferred_element_type=jnp.float32) + ba2_ref[0, :])

    @pl.when(pl.program_id(0) == 0)
    def _():
        zc = jax.nn.relu(jnp.dot(p, wc1_ref[...], preferred_element_type=jnp.float32)
                         + bc1_ref[0, :])
        val_ref[...] = (jnp.dot(zc, wc2_ref[...], preferred_element_type=jnp.float32)
                        + bc2_ref[0, :])


def _heads(pooled, Wa1, ba1, Wa2, ba2, Wc1, bc1, Wc2, bc2):
    NE = Wa2.shape[1]
    CBLK = 12800
    action, value = pl.pallas_call(
        _heads_body,
        grid=(NE // CBLK,),
        in_specs=[
            pl.BlockSpec((NG, HID), lambda j: (0, 0)),
            pl.BlockSpec((HID, HID), lambda j: (0, 0)),
            pl.BlockSpec((1, HID), lambda j: (0, 0)),
            pl.BlockSpec((HID, CBLK), lambda j: (0, j)),
            pl.BlockSpec((1, CBLK), lambda j: (0, j)),
            pl.BlockSpec((HID, HID), lambda j: (0, 0)),
            pl.BlockSpec((1, HID), lambda j: (0, 0)),
            pl.BlockSpec((HID, 8), lambda j: (0, 0)),
            pl.BlockSpec((1, 8), lambda j: (0, 0)),
        ],
        out_specs=[
            pl.BlockSpec((NG, CBLK), lambda j: (0, j)),
            pl.BlockSpec((NG, 8), lambda j: (0, 0)),
        ],
        out_shape=[
            jax.ShapeDtypeStruct((NG, NE), jnp.float32),
            jax.ShapeDtypeStruct((NG, 8), jnp.float32),
        ],
    )(pooled, Wa1, ba1[None, :], Wa2, ba2[None, :], Wc1, bc1[None, :],
      jnp.pad(Wc2, ((0, 0), (0, 7))), jnp.pad(bc2, (0, 7))[None, :])
    return action, value[:, :1]


def kernel(x, edge_index, batch, W1, a1s, a1d, b1, W2, a2s, a2d, b2,
           W3, a3s, a3d, b3, Wa1, ba1, Wa2, ba2, Wc1, bc1, Wc2, bc2):
    loop = jnp.arange(N, dtype=edge_index.dtype)
    src = jnp.concatenate([edge_index[0], loop])
    dst = jnp.concatenate([edge_index[1], loop])
    perm = jnp.argsort(dst)
    ss = src[perm].astype(jnp.int32)
    ds = dst[perm]
    nptr = jnp.searchsorted(ds, jnp.arange(N + 1, dtype=jnp.int32)).astype(jnp.int32)
    nptr_pad = jnp.concatenate(
        [nptr, jnp.full((NPTR_PAD - (N + 1),), ETOT, jnp.int32)])
    ss_pad = jnp.concatenate([ss, jnp.zeros((SS_PAD - ETOT,), jnp.int32)])

    h, sa = _gat_project(x, W1, a1s, a1d, 4)
    h = _gat_edge_sc(h, sa, ss_pad, nptr_pad, b1, 4)
    h, sa = _gat_project(h, W2, a2s, a2d, 4)
    h = _gat_edge_sc(h, sa, ss_pad, nptr_pad, b2, 4)
    h, sa = _gat_project(h, W3, a3s, a3d, 1)
    h = _gat_edge_sc(h, sa, ss_pad, nptr_pad, b3, 1)

    pooled = _pool(batch, h)
    return _heads(pooled, Wa1, ba1, Wa2, ba2, Wc1, bc1, Wc2, bc2)
